# baseline (speedup 1.0000x reference)
import functools

import jax
import jax.numpy as jnp
from jax import lax
from jax.experimental import pallas as pl
from jax.experimental.pallas import tpu as pltpu

_D = 1024
_V = 8192


def _tc_gather(idx, table3):
    n_tc = idx.shape[0]
    grid = 32
    rows_per = n_tc // grid

    def body(idx_ref, table_ref, out_ref):
        g = pl.program_id(0)

        def row(j, carry):
            i = idx_ref[g * rows_per + j]
            out_ref[j] = table_ref[i]
            return carry

        lax.fori_loop(0, rows_per, row, 0, unroll=8)

    return pl.pallas_call(
        body,
        grid_spec=pltpu.PrefetchScalarGridSpec(
            num_scalar_prefetch=1,
            grid=(grid,),
            in_specs=[
                pl.BlockSpec((_V, 8, 128), lambda g, idx_ref: (0, 0, 0)),
            ],
            out_specs=pl.BlockSpec((rows_per, 8, 128), lambda g, idx_ref: (g, 0, 0)),
        ),
        out_shape=jax.ShapeDtypeStruct((n_tc, 8, 128), jnp.float32),
    )(idx, table3)


def kernel(token_positions, wpe):
    n = token_positions.size
    idx = token_positions.reshape(n).astype(jnp.int32)
    table3 = wpe.reshape(_V, 8, 128)
    out = _tc_gather(idx, table3)
    return out.reshape(token_positions.shape + (wpe.shape[-1],))
